# Initial kernel scaffold; baseline (speedup 1.0000x reference)
import jax
import jax.numpy as jnp
from jax.experimental import pallas as pl


def kernel(images, anchors, all_gt_bboxes, all_gt_orig_classes):
    raise NotImplementedError("write your pallas kernel here")


# speedup vs baseline: 29.9903x; 29.9903x over previous
import functools

import jax
import jax.numpy as jnp
from jax.experimental import pallas as pl
from jax.experimental.pallas import tpu as pltpu

_B, _N, _G = 8, 20000, 32
_R, _C = 160, 125
_POS_T, _NEG_T = 0.7, 0.3
_MAX_SAMPLES, _N_POS_MAX = 256, 128


def _rpn_body(a_ref, gt_ref, conf_ref, off_ref):
    ax1 = a_ref[0, 0]
    ay1 = a_ref[0, 1]
    ax2 = a_ref[0, 2]
    ay2 = a_ref[0, 3]
    area_a = (ax2 - ax1) * (ay2 - ay1)

    best = jnp.full((_R, _C), -1.0, jnp.float32)
    t1 = jnp.zeros((_R, _C), jnp.float32)
    t2 = jnp.zeros((_R, _C), jnp.float32)
    t3 = jnp.zeros((_R, _C), jnp.float32)
    t4 = jnp.zeros((_R, _C), jnp.float32)
    ious = []
    bpg = []
    for g in range(_G):
        gx1 = gt_ref[0, 0, 4 * g + 0]
        gy1 = gt_ref[0, 0, 4 * g + 1]
        gx2 = gt_ref[0, 0, 4 * g + 2]
        gy2 = gt_ref[0, 0, 4 * g + 3]
        lt_x = jnp.maximum(ax1, gx1)
        lt_y = jnp.maximum(ay1, gy1)
        rb_x = jnp.minimum(ax2, gx2)
        rb_y = jnp.minimum(ay2, gy2)
        w = jnp.maximum(rb_x - lt_x, 0.0)
        h = jnp.maximum(rb_y - lt_y, 0.0)
        inter = w * h
        area_g = (gx2 - gx1) * (gy2 - gy1)
        union = (area_a + area_g) - inter
        iou = inter / jnp.maximum(union, 1e-9)
        ious.append(iou)
        bpg.append(jnp.max(iou))
        m = iou > best
        best = jnp.where(m, iou, best)
        t1 = jnp.where(m, gx1, t1)
        t2 = jnp.where(m, gy1, t2)
        t3 = jnp.where(m, gx2, t3)
        t4 = jnp.where(m, gy2, t4)

    conf = jnp.where(best < _NEG_T, 0, -1).astype(jnp.int32)
    conf = jnp.where(best >= _POS_T, 1, conf)
    force = jnp.zeros((_R, _C), jnp.bool_)
    for g in range(_G):
        force = force | ((ious[g] >= bpg[g]) & (bpg[g] > 0.0))
    conf = jnp.where(force, 1, conf)

    pos = conf == 1
    neg = conf == 0

    ri = jax.lax.broadcasted_iota(jnp.int32, (_C, _C), 0)
    ci = jax.lax.broadcasted_iota(jnp.int32, (_C, _C), 1)
    tri = (ri <= ci).astype(jnp.bfloat16)
    rr = jax.lax.broadcasted_iota(jnp.int32, (_R, _R), 0)
    rc = jax.lax.broadcasted_iota(jnp.int32, (_R, _R), 1)
    stri = (rc < rr).astype(jnp.bfloat16)

    def cumsum2d(mask):
        cs = jnp.dot(mask.astype(jnp.bfloat16), tri,
                     preferred_element_type=jnp.float32)
        rowtot = cs[:, _C - 1:_C]
        pref = jnp.dot(stri, rowtot.astype(jnp.bfloat16),
                       preferred_element_type=jnp.float32)
        return cs + pref

    cpos = cumsum2d(pos)
    cneg = cumsum2d(neg)
    tot_pos = cpos[_R - 1:_R, _C - 1:_C]
    tot_neg = cneg[_R - 1:_R, _C - 1:_C]
    npos_keep = jnp.minimum(tot_pos, float(_N_POS_MAX))
    nneg_keep = jnp.minimum(tot_neg, float(_MAX_SAMPLES) - npos_keep)
    keep_pos = pos & (cpos <= npos_keep)
    keep_neg = neg & (cneg <= nneg_keep)
    remove = (pos & ~keep_pos) | (neg & ~keep_neg)

    conf = jnp.where(remove, -1, conf)
    g0x1 = gt_ref[0, 0, 0]
    g0y1 = gt_ref[0, 0, 1]
    g0x2 = gt_ref[0, 0, 2]
    g0y2 = gt_ref[0, 0, 3]
    t1 = jnp.where(remove, g0x1, t1)
    t2 = jnp.where(remove, g0y1, t2)
    t3 = jnp.where(remove, g0x2, t3)
    t4 = jnp.where(remove, g0y2, t4)

    t_cx = (t1 + t3) * 0.5
    t_cy = (t2 + t4) * 0.5
    t_w = t3 - t1
    t_h = t4 - t2
    a_cx = (ax1 + ax2) * 0.5
    a_cy = (ay1 + ay2) * 0.5
    a_w = ax2 - ax1
    a_h = ay2 - ay1

    conf_ref[0] = conf
    off_ref[0, 0] = (t_cx - a_cx) / a_w
    off_ref[0, 1] = (t_cy - a_cy) / a_h
    off_ref[0, 2] = jnp.log(t_w / a_w)
    off_ref[0, 3] = jnp.log(t_h / a_h)


def kernel(images, anchors, all_gt_bboxes, all_gt_orig_classes):
    del images, all_gt_orig_classes
    a_t = anchors.transpose(0, 2, 1).reshape(_B, 4, _R, _C)
    gt_s = all_gt_bboxes.reshape(_B, 1, 4 * _G)
    conf3, off4 = pl.pallas_call(
        _rpn_body,
        grid=(_B,),
        in_specs=[
            pl.BlockSpec((1, 4, _R, _C), lambda b: (b, 0, 0, 0)),
            pl.BlockSpec((1, 1, 4 * _G), lambda b: (b, 0, 0),
                         memory_space=pltpu.SMEM),
        ],
        out_specs=[
            pl.BlockSpec((1, _R, _C), lambda b: (b, 0, 0)),
            pl.BlockSpec((1, 4, _R, _C), lambda b: (b, 0, 0, 0)),
        ],
        out_shape=[
            jax.ShapeDtypeStruct((_B, _R, _C), jnp.int32),
            jax.ShapeDtypeStruct((_B, 4, _R, _C), jnp.float32),
        ],
    )(a_t, gt_s)
    conf = conf3.reshape(_B, _N)
    off = off4.reshape(_B, 4, _N).transpose(0, 2, 1)
    return conf, off


# speedup vs baseline: 30.0365x; 1.0015x over previous
import functools

import jax
import jax.numpy as jnp
from jax.experimental import pallas as pl
from jax.experimental.pallas import tpu as pltpu

_B, _N, _G = 8, 20000, 32
_R, _C = 160, 125
_POS_T, _NEG_T = 0.7, 0.3
_MAX_SAMPLES, _N_POS_MAX = 256, 128


def _rpn_body(a_ref, gt_ref, conf_ref, off_ref):
    ax1 = a_ref[0, 0]
    ay1 = a_ref[0, 1]
    ax2 = a_ref[0, 2]
    ay2 = a_ref[0, 3]
    area_a = (ax2 - ax1) * (ay2 - ay1)

    best = jnp.full((_R, _C), -1.0, jnp.float32)
    t1 = jnp.zeros((_R, _C), jnp.float32)
    t2 = jnp.zeros((_R, _C), jnp.float32)
    t3 = jnp.zeros((_R, _C), jnp.float32)
    t4 = jnp.zeros((_R, _C), jnp.float32)
    force = jnp.zeros((_R, _C), jnp.bool_)
    for g in range(_G):
        gx1 = gt_ref[0, 0, 4 * g + 0]
        gy1 = gt_ref[0, 0, 4 * g + 1]
        gx2 = gt_ref[0, 0, 4 * g + 2]
        gy2 = gt_ref[0, 0, 4 * g + 3]
        lt_x = jnp.maximum(ax1, gx1)
        lt_y = jnp.maximum(ay1, gy1)
        rb_x = jnp.minimum(ax2, gx2)
        rb_y = jnp.minimum(ay2, gy2)
        w = jnp.maximum(rb_x - lt_x, 0.0)
        h = jnp.maximum(rb_y - lt_y, 0.0)
        inter = w * h
        area_g = (gx2 - gx1) * (gy2 - gy1)
        union = (area_a + area_g) - inter
        iou = inter / jnp.maximum(union, 1e-9)
        bpg = jnp.max(iou)
        force = force | ((iou >= bpg) & (bpg > 0.0))
        m = iou > best
        best = jnp.where(m, iou, best)
        t1 = jnp.where(m, gx1, t1)
        t2 = jnp.where(m, gy1, t2)
        t3 = jnp.where(m, gx2, t3)
        t4 = jnp.where(m, gy2, t4)

    conf = jnp.where(best < _NEG_T, 0, -1).astype(jnp.int32)
    conf = jnp.where(best >= _POS_T, 1, conf)
    conf = jnp.where(force, 1, conf)

    pos = conf == 1
    neg = conf == 0

    ri = jax.lax.broadcasted_iota(jnp.int32, (_C, _C), 0)
    ci = jax.lax.broadcasted_iota(jnp.int32, (_C, _C), 1)
    tri = (ri <= ci).astype(jnp.bfloat16)
    rr = jax.lax.broadcasted_iota(jnp.int32, (_R, _R), 0)
    rc = jax.lax.broadcasted_iota(jnp.int32, (_R, _R), 1)
    stri = (rc < rr).astype(jnp.bfloat16)

    def cumsum2d(mask):
        cs = jnp.dot(mask.astype(jnp.bfloat16), tri,
                     preferred_element_type=jnp.float32)
        rowtot = cs[:, _C - 1:_C]
        pref = jnp.dot(stri, rowtot.astype(jnp.bfloat16),
                       preferred_element_type=jnp.float32)
        return cs + pref

    cpos = cumsum2d(pos)
    cneg = cumsum2d(neg)
    tot_pos = cpos[_R - 1:_R, _C - 1:_C]
    tot_neg = cneg[_R - 1:_R, _C - 1:_C]
    npos_keep = jnp.minimum(tot_pos, float(_N_POS_MAX))
    nneg_keep = jnp.minimum(tot_neg, float(_MAX_SAMPLES) - npos_keep)
    keep_pos = pos & (cpos <= npos_keep)
    keep_neg = neg & (cneg <= nneg_keep)
    remove = (pos & ~keep_pos) | (neg & ~keep_neg)

    conf = jnp.where(remove, -1, conf)
    g0x1 = gt_ref[0, 0, 0]
    g0y1 = gt_ref[0, 0, 1]
    g0x2 = gt_ref[0, 0, 2]
    g0y2 = gt_ref[0, 0, 3]
    t1 = jnp.where(remove, g0x1, t1)
    t2 = jnp.where(remove, g0y1, t2)
    t3 = jnp.where(remove, g0x2, t3)
    t4 = jnp.where(remove, g0y2, t4)

    t_cx = (t1 + t3) * 0.5
    t_cy = (t2 + t4) * 0.5
    t_w = t3 - t1
    t_h = t4 - t2
    a_cx = (ax1 + ax2) * 0.5
    a_cy = (ay1 + ay2) * 0.5
    a_w = ax2 - ax1
    a_h = ay2 - ay1

    conf_ref[0] = conf
    off_ref[0, 0] = (t_cx - a_cx) / a_w
    off_ref[0, 1] = (t_cy - a_cy) / a_h
    off_ref[0, 2] = jnp.log(t_w / a_w)
    off_ref[0, 3] = jnp.log(t_h / a_h)


def kernel(images, anchors, all_gt_bboxes, all_gt_orig_classes):
    del images, all_gt_orig_classes
    a_t = anchors.transpose(0, 2, 1).reshape(_B, 4, _R, _C)
    gt_s = all_gt_bboxes.reshape(_B, 1, 4 * _G)
    conf3, off4 = pl.pallas_call(
        _rpn_body,
        grid=(_B,),
        in_specs=[
            pl.BlockSpec((1, 4, _R, _C), lambda b: (b, 0, 0, 0)),
            pl.BlockSpec((1, 1, 4 * _G), lambda b: (b, 0, 0),
                         memory_space=pltpu.SMEM),
        ],
        out_specs=[
            pl.BlockSpec((1, _R, _C), lambda b: (b, 0, 0)),
            pl.BlockSpec((1, 4, _R, _C), lambda b: (b, 0, 0, 0)),
        ],
        out_shape=[
            jax.ShapeDtypeStruct((_B, _R, _C), jnp.int32),
            jax.ShapeDtypeStruct((_B, 4, _R, _C), jnp.float32),
        ],
    )(a_t, gt_s)
    conf = conf3.reshape(_B, _N)
    off = off4.reshape(_B, 4, _N).transpose(0, 2, 1)
    return conf, off
